# baseline (speedup 1.0000x reference)
import functools

import jax
import jax.numpy as jnp
from jax import lax
from jax.experimental import pallas as pl
from jax.experimental.pallas import tpu as pltpu
from jax.experimental.pallas import tpu_sc as plsc

N = 10000
E = 320000
H = 128
NUM_ROUNDS = 6

NW = 32
NPW = 320
NPAD = NW * NPW
CH = 128
EPAD = E + 3 * CH
PF_W = 2576
PT_W = 2896
CT_G = 5248
PT_G = 16384


def _sc_agg_body(h, srcp, eoffp, cbp, poffg, ptabg, c0i, c1i,
                 out_s, out_m, out_c0, out_c1,
                 eoff_v, cb_v, idx_v, rows_v, poff_v, ptab_v, osum_v, omax_v,
                 sem):
    c = lax.axis_index("c")
    s = lax.axis_index("s")
    w = c * 16 + s
    n0 = w * NPW

    pltpu.sync_copy(eoffp, eoff_v)
    pltpu.sync_copy(cbp, cb_v)
    ev = eoff_v[pl.ds(w, 16)]
    e0 = ev[0]
    e0a = (e0 // 8) * 8
    cv = cb_v[pl.ds(w, 16)]
    cb = cv[0]
    nch = cv[1] - cb

    cba = (cb // 8) * 8
    dcb = cb - cba
    pltpu.sync_copy(poffg.at[pl.ds(cba, PF_W)], poff_v)
    pv = poff_v[pl.ds(dcb, 16)]
    pb = pv[0]
    pba = (pb // 8) * 8
    dpb = pb - pba
    pltpu.sync_copy(ptabg.at[pl.ds(pba, PT_W)], ptab_v)

    def fire(g):
        slot = (g % 2) * CH
        pltpu.sync_copy(srcp.at[pl.ds(e0a + g * CH, CH)],
                        idx_v.at[pl.ds(slot, CH)])
        pltpu.async_copy(h.at[idx_v.at[pl.ds(slot, CH)]],
                         rows_v.at[pl.ds(slot, CH)], sem)

    def wait_one():
        pltpu.make_async_copy(h.at[pl.ds(0, CH)],
                              rows_v.at[pl.ds(0, CH)], sem).wait()

    zf = jnp.zeros((16,), jnp.float32)
    ninf = jnp.full((16,), -jnp.inf, jnp.float32)

    def init_body(i, carry):
        for j in range(8):
            osum_v[i, pl.ds(j * 16, 16)] = zf
            omax_v[i, pl.ds(j * 16, 16)] = ninf
        return carry

    lax.fori_loop(0, NPW, init_body, 0)

    fire(0)
    fire(1)

    def chunk_body(g, carry):
        s8, m8 = carry
        wait_one()
        pvec = poff_v[pl.ds(dcb + g, 16)]
        p0 = pvec[0]
        p1 = pvec[1]

        def piece_body(k, carry2):
            ss, mm = carry2
            fvec = ptab_v[pl.ds(dpb + (p0 - pb) + k, 16)]
            f = fvec[0]
            take = f & 0xFF
            row0 = (f >> 8) & 0xFF
            ln = (f >> 16) & 0x1FF
            flush = (f >> 26) & 1

            def inner(e, sm):
                ss3, mm3 = sm
                r = row0 + e
                ss4 = []
                mm4 = []
                for j in range(8):
                    row = rows_v[r, pl.ds(j * 16, 16)]
                    ss4.append(ss3[j] + row)
                    mm4.append(jnp.maximum(mm3[j], row))
                return tuple(ss4), tuple(mm4)

            ss, mm = lax.fori_loop(0, take, inner, (ss, mm))
            keepf = jnp.full((16,), 1.0, jnp.float32) * (1 - flush).astype(
                jnp.float32)
            clamp = (keepf * 2.0 - 1.0) * jnp.full((16,), jnp.inf, jnp.float32)
            ss2 = []
            mm2 = []
            for j in range(8):
                osum_v[ln, pl.ds(j * 16, 16)] = ss[j]
                omax_v[ln, pl.ds(j * 16, 16)] = mm[j]
                ss2.append(ss[j] * keepf)
                mm2.append(jnp.minimum(mm[j], clamp))
            return tuple(ss2), tuple(mm2)

        s8, m8 = lax.fori_loop(0, p1 - p0, piece_body, (s8, m8))
        fire(g + 2)
        return (s8, m8)

    lax.fori_loop(0, nch, chunk_body, ((zf,) * 8, (ninf,) * 8))

    wait_one()
    wait_one()

    pltpu.sync_copy(osum_v, out_s.at[pl.ds(n0, NPW)])
    pltpu.sync_copy(omax_v, out_m.at[pl.ds(n0, NPW)])

    cidx = idx_v.at[pl.ds(0, 64)]
    crow = rows_v.at[pl.ds(0, 64)]
    for t in range(NPW // 64):
        off = n0 + t * 64
        pltpu.sync_copy(c0i.at[pl.ds(off, 64)], cidx)
        pltpu.async_copy(h.at[cidx], crow, sem).wait()
        pltpu.sync_copy(crow, out_c0.at[pl.ds(off, 64)])
        pltpu.sync_copy(c1i.at[pl.ds(off, 64)], cidx)
        pltpu.async_copy(h.at[cidx], crow, sem).wait()
        pltpu.sync_copy(crow, out_c1.at[pl.ds(off, 64)])


_f32 = jnp.float32


@functools.cache
def _sc_agg_built():
    return functools.partial(
        pl.kernel,
        out_type=[jax.ShapeDtypeStruct((NPAD, H), _f32)] * 4,
        mesh=plsc.VectorSubcoreMesh(core_axis_name="c", subcore_axis_name="s"),
        scratch_types=[
            pltpu.VMEM((48,), jnp.int32),
            pltpu.VMEM((48,), jnp.int32),
            pltpu.VMEM((2 * CH,), jnp.int32),
            pltpu.VMEM((2 * CH, H), _f32),
            pltpu.VMEM((PF_W,), jnp.int32),
            pltpu.VMEM((PT_W,), jnp.int32),
            pltpu.VMEM((NPW, H), _f32),
            pltpu.VMEM((NPW, H), _f32),
            pltpu.SemaphoreType.DMA,
        ],
    )(_sc_agg_body)


def _sc_agg(*args):
    return _sc_agg_built()(*args)


BR = 1000


def _mm(a, b):
    return lax.dot_general(a, b, (((1,), (0,)), ((), ())),
                           precision=lax.Precision.DEFAULT,
                           preferred_element_type=_f32)


def _sigmoid(x):
    return 1.0 / (1.0 + jnp.exp(-x))


def _gru(x, h, wih, whh, bih, bhh):
    gi = _mm(x, wih) + bih
    gh = _mm(h, whh) + bhh
    r = _sigmoid(gi[:, 0:H] + gh[:, 0:H])
    z = _sigmoid(gi[:, H:2 * H] + gh[:, H:2 * H])
    nn_ = jnp.tanh(gi[:, 2 * H:3 * H] + r * gh[:, 2 * H:3 * H])
    return (1.0 - z) * nn_ + z * h


def _tc_round_body(h, sm, mx, c0, c1, meta, acw, acb, abw, abb,
                   wihc, whhc, bihc, bhhc, wihb, whhb, bihb, bhhb,
                   wihl, whhl, bihl, bhhl, lng, lnb, hout):
    cnt = meta[:, 0:1]
    invd = meta[:, 1:2]
    il = meta[:, 2:3]
    ic = meta[:, 3:4]
    ip = meta[:, 4:5]
    hv = h[...]
    mean = sm[...] * invd
    mxv = jnp.where(cnt > 0.0, mx[...], 0.0)
    aggc = _mm(jnp.concatenate([mean, mxv], axis=1), acw[...]) + acb[...]
    aggb = _mm(jnp.concatenate([c0[...], c1[...]], axis=1), abw[...]) + abb[...]
    agg = jnp.where(il > 0.0, 0.0,
                    jnp.where(ic > 0.0, aggc,
                              jnp.where(ip > 0.0, aggb, mean)))
    uc = _gru(agg, hv, wihc[...], whhc[...], bihc[...], bhhc[...])
    ub = _gru(agg, hv, wihb[...], whhb[...], bihb[...], bhhb[...])
    ul = _gru(agg, hv, wihl[...], whhl[...], bihl[...], bhhl[...])
    upd = jnp.where(il > 0.0, ul, jnp.where(ip > 0.0, ub, uc))
    x = upd + hv
    mu = jnp.mean(x, axis=1, keepdims=True)
    var = jnp.mean((x - mu) * (x - mu), axis=1, keepdims=True)
    hout[...] = (x - mu) * lax.rsqrt(var + 1e-5) * lng[...] + lnb[...]


def _node_spec():
    return pl.BlockSpec((BR, H), lambda i: (i, 0))


def _full_spec(shape):
    return pl.BlockSpec(shape, lambda i: tuple(0 for _ in shape))


def _tc_round(h, sm, mx, c0, c1, meta, *weights):
    wspecs = [_full_spec(w.shape) for w in weights]
    return pl.pallas_call(
        _tc_round_body,
        grid=(N // BR,),
        in_specs=[_node_spec()] * 5 + [pl.BlockSpec((BR, 8), lambda i: (i, 0))]
                 + wspecs,
        out_specs=_node_spec(),
        out_shape=jax.ShapeDtypeStruct((N, H), _f32),
    )(h, sm, mx, c0, c1, meta, *weights)


def _tc_init_body(meta, ttab, ctab, vtab, iw, ib, hout):
    nt = meta[:, 0:1]
    ci = meta[:, 1:2]
    vi = meta[:, 2:3]
    oh_t = (nt == lax.broadcasted_iota(jnp.int32, (BR, 8), 1).astype(_f32)
            ).astype(_f32)
    te = _mm(oh_t, ttab[...])
    oh_c = (ci == lax.broadcasted_iota(jnp.int32, (BR, 24), 1).astype(_f32)
            ).astype(_f32)
    ce = _mm(oh_c, ctab[...])
    oh_v = (vi == lax.broadcasted_iota(jnp.int32, (BR, 8), 1).astype(_f32)
            ).astype(_f32)
    vee = _mm(oh_v, vtab[...])
    ve = jnp.where(nt == 3.0, ce, jnp.where(nt == 4.0, vee, 0.0))
    hout[...] = _mm(jnp.concatenate([te, ve], axis=1), iw[...]) + ib[...]


def _tc_init(meta, ttab, ctab, vtab, iw, ib):
    specs = [pl.BlockSpec((BR, 8), lambda i: (i, 0))]
    specs += [_full_spec(x.shape) for x in (ttab, ctab, vtab, iw, ib)]
    return pl.pallas_call(
        _tc_init_body,
        grid=(N // BR,),
        in_specs=specs,
        out_specs=_node_spec(),
        out_shape=jax.ShapeDtypeStruct((N, H), _f32),
    )(meta, ttab, ctab, vtab, iw, ib)


def _tc_final_body(h, gw, gb, emb, gco, gc_v):
    i = pl.program_id(0)

    @pl.when(i == 0)
    def _():
        gc_v[...] = _mm(h[0:8, :], gw[...]) + gb[...]

    hv = h[...]
    emb[:, 0:H] = hv
    emb[:, H:2 * H] = jnp.broadcast_to(gc_v[0:1, :], (BR, H))
    gco[...] = gc_v[...]


def _tc_final(h, gw, gb):
    return pl.pallas_call(
        _tc_final_body,
        grid=(N // BR,),
        in_specs=[_node_spec(), _full_spec((H, H)), _full_spec((1, H))],
        out_specs=[pl.BlockSpec((BR, 2 * H), lambda i: (i, 0)),
                   _full_spec((8, H))],
        out_shape=[jax.ShapeDtypeStruct((N, 2 * H), _f32),
                   jax.ShapeDtypeStruct((8, H), _f32)],
        scratch_shapes=[pltpu.VMEM((8, H), _f32)],
    )(h, gw, gb)


def _edge_tables(src, dst):
    i32 = jnp.int32
    start = jnp.searchsorted(dst, jnp.arange(N + 1, dtype=i32), side='left')
    start = start.astype(i32)
    bounds = jnp.minimum(jnp.arange(33, dtype=i32) * NPW, N)
    eoff = start[bounds]
    eoffp = jnp.concatenate([eoff, jnp.full((15,), E, i32)])
    srcp = jnp.concatenate([src, jnp.zeros((3 * CH,), i32)])

    w_of_e = dst // NPW
    e0_e = eoff[w_of_e]
    a_e = e0_e % 8
    eidx = jnp.arange(E, dtype=i32)
    prev = jnp.concatenate([jnp.full((1,), -1, i32), dst[:E - 1]])
    newp = (dst != prev) | (eidx == e0_e) | (((eidx - e0_e + a_e) % CH) == 0)
    pid = jnp.cumsum(newp.astype(i32)).astype(i32) - 1
    ptot = pid[E - 1] + 1
    sp = jnp.searchsorted(pid, jnp.arange(PT_G + 1, dtype=i32),
                          side='left').astype(i32)
    take = sp[1:] - sp[:PT_G]
    ef = jnp.clip(sp[:PT_G], 0, E - 1)
    dstf = dst[ef]
    lnode = dstf % NPW
    e0f = eoff[dstf // NPW]
    e0af = e0f - e0f % 8
    row = (((ef - e0af) // CH) % 2) * CH + (ef - e0af) % CH
    nexte = jnp.clip(sp[1:], 0, E - 1)
    flush = ((sp[1:] >= E) | (dst[nexte] != dstf)).astype(i32)
    ptabg = ((take & 0xFF) | ((row & 0xFF) << 8) | ((lnode & 0x1FF) << 16)
             | (flush << 26))

    e0w = eoff[:32]
    e0aw = e0w - e0w % 8
    nchw = (eoff[1:] - e0aw + CH - 1) // CH
    cbase = jnp.concatenate([jnp.zeros((1,), i32),
                             jnp.cumsum(nchw).astype(i32)])
    cbp = jnp.concatenate([cbase, jnp.full((15,), cbase[32], i32)])
    cq = jnp.arange(CT_G, dtype=i32)
    wq = jnp.clip(jnp.searchsorted(cbase, cq, side='right').astype(i32) - 1,
                  0, 31)
    e0q = eoff[wq]
    sposq = jnp.maximum(e0q, (e0q - e0q % 8) + (cq - cbase[wq]) * CH)
    poffg = jnp.where(cq >= cbase[32], ptot,
                      pid[jnp.clip(sposq, 0, E - 1)]).astype(i32)
    return start, eoffp, srcp, cbp, poffg, ptabg


def kernel(node_type, coeff_idx, var_idx, src, dst, type_table, coeff_table,
           var_table, init_w, init_b, agg_comm_w, agg_comm_b, agg_bin_w,
           agg_bin_b, wih_comm, whh_comm, bih_comm, bhh_comm, wih_bin,
           whh_bin, bih_bin, bhh_bin, wih_leaf, whh_leaf, bih_leaf, bhh_leaf,
           ln_g, ln_b, glob_w, glob_b):
    i32 = jnp.int32
    src = src.astype(i32)
    dst = dst.astype(i32)

    start, eoffp, srcp, cbp, poffg, ptabg = _edge_tables(src, dst)
    counts = (start[1:] - start[:N]).astype(_f32)
    c0 = src[jnp.clip(start[:N], 0, E - 1)]
    c1 = src[jnp.clip(start[:N] + 1, 0, E - 1)]
    c0p = jnp.concatenate([c0, jnp.zeros((NPAD - N,), i32)])
    c1p = jnp.concatenate([c1, jnp.zeros((NPAD - N,), i32)])

    is_leaf = (counts == 0.0)
    is_comm = (node_type <= 1) & (~is_leaf)
    is_pow2 = (node_type == 2) & (counts == 2.0)
    invd = 1.0 / jnp.maximum(counts, 1.0)
    meta = jnp.stack([counts, invd,
                      is_leaf.astype(_f32), is_comm.astype(_f32),
                      is_pow2.astype(_f32),
                      jnp.zeros((N,), _f32), jnp.zeros((N,), _f32),
                      jnp.zeros((N,), _f32)], axis=1)

    meta_init = jnp.stack([node_type.astype(_f32), coeff_idx.astype(_f32),
                           var_idx.astype(_f32)] + [jnp.zeros((N,), _f32)] * 5,
                          axis=1)
    ttab = jnp.zeros((8, H), _f32).at[:6].set(type_table)
    ctab = jnp.zeros((24, H), _f32).at[:19].set(coeff_table)
    vtab = jnp.zeros((8, H), _f32).at[:4].set(var_table)

    rw = [agg_comm_w, agg_comm_b.reshape(1, H), agg_bin_w,
          agg_bin_b.reshape(1, H),
          wih_comm, whh_comm, bih_comm.reshape(1, 3 * H),
          bhh_comm.reshape(1, 3 * H),
          wih_bin, whh_bin, bih_bin.reshape(1, 3 * H),
          bhh_bin.reshape(1, 3 * H),
          wih_leaf, whh_leaf, bih_leaf.reshape(1, 3 * H),
          bhh_leaf.reshape(1, 3 * H),
          ln_g.reshape(1, H), ln_b.reshape(1, H)]

    h = _tc_init(meta_init, ttab, ctab, vtab, init_w, init_b.reshape(1, H))
    for _r in range(NUM_ROUNDS):
        sm, mx, hc0, hc1 = _sc_agg(h, srcp, eoffp, cbp, poffg, ptabg,
                                   c0p, c1p)
        h = _tc_round(h, sm, mx, hc0, hc1, meta, *rw)

    emb, gc8 = _tc_final(h, glob_w, glob_b.reshape(1, H))
    return emb, gc8[0]


# speedup vs baseline: 2.7701x; 2.7701x over previous
import functools

import jax
import jax.numpy as jnp
from jax import lax
from jax.experimental import pallas as pl
from jax.experimental.pallas import tpu as pltpu
from jax.experimental.pallas import tpu_sc as plsc

N = 10000
E = 320000
H = 128
NUM_ROUNDS = 6

NW = 32
NPW = 320
NPAD = NW * NPW
CH = 128
EPAD = E + 3 * CH
PF_W = 2576
PT_W = 2896
CT_G = 5248
PT_G = 16384


def _sc_agg_body(h, srcp, eoffp, cbp, poffg, ptabg, c0i, c1i,
                 out_s, out_m, out_c0, out_c1,
                 eoff_v, cb_v, idx_v, rows_v, poff_v, ptab_v, osum_v, omax_v,
                 sem):
    c = lax.axis_index("c")
    s = lax.axis_index("s")
    w = c * 16 + s
    n0 = w * NPW

    pltpu.sync_copy(eoffp, eoff_v)
    pltpu.sync_copy(cbp, cb_v)
    ev = eoff_v[pl.ds(w, 16)]
    e0 = ev[0]
    e0a = (e0 // 8) * 8
    cv = cb_v[pl.ds(w, 16)]
    cb = cv[0]
    nch = cv[1] - cb

    cba = (cb // 8) * 8
    dcb = cb - cba
    pltpu.sync_copy(poffg.at[pl.ds(cba, PF_W)], poff_v)
    pv = poff_v[pl.ds(dcb, 16)]
    pb = pv[0]
    pba = (pb // 8) * 8
    dpb = pb - pba
    pltpu.sync_copy(ptabg.at[pl.ds(pba, PT_W)], ptab_v)

    def fire(g):
        slot = (g % 2) * CH
        pltpu.sync_copy(srcp.at[pl.ds(e0a + g * CH, CH)],
                        idx_v.at[pl.ds(slot, CH)])
        pltpu.async_copy(h.at[idx_v.at[pl.ds(slot, CH)]],
                         rows_v.at[pl.ds(slot, CH)], sem)

    def wait_one():
        pltpu.make_async_copy(h.at[pl.ds(0, CH)],
                              rows_v.at[pl.ds(0, CH)], sem).wait()

    fire(0)
    fire(1)

    zf = jnp.zeros((16,), jnp.float32)
    ninf = jnp.full((16,), -jnp.inf, jnp.float32)

    def init_body(i, carry):
        for j in range(8):
            osum_v[i, pl.ds(j * 16, 16)] = zf
            omax_v[i, pl.ds(j * 16, 16)] = ninf
        return carry

    lax.fori_loop(0, NPW, init_body, 0)

    def chunk_body(g, carry):
        s8, m8 = carry
        wait_one()
        pvec = poff_v[pl.ds(dcb + g, 16)]
        p0 = pvec[0]
        p1 = pvec[1]

        def piece_body(k, carry2):
            ss, mm = carry2
            fvec = ptab_v[pl.ds(dpb + (p0 - pb) + k, 16)]
            f = fvec[0]
            take = f & 0xFF
            row0 = (f >> 8) & 0xFF
            ln = (f >> 16) & 0x1FF
            flush = (f >> 26) & 1

            def inner(e, sm):
                ss3, mm3 = sm
                r = row0 + e
                ss4 = []
                mm4 = []
                for j in range(8):
                    row = rows_v[r, pl.ds(j * 16, 16)]
                    ss4.append(ss3[j] + row)
                    mm4.append(jnp.maximum(mm3[j], row))
                return tuple(ss4), tuple(mm4)

            ss, mm = lax.fori_loop(0, take, inner, (ss, mm))
            keepf = jnp.full((16,), 1.0, jnp.float32) * (1 - flush).astype(
                jnp.float32)
            clamp = (keepf * 2.0 - 1.0) * jnp.full((16,), jnp.inf, jnp.float32)
            ss2 = []
            mm2 = []
            for j in range(8):
                osum_v[ln, pl.ds(j * 16, 16)] = ss[j]
                omax_v[ln, pl.ds(j * 16, 16)] = mm[j]
                ss2.append(ss[j] * keepf)
                mm2.append(jnp.minimum(mm[j], clamp))
            return tuple(ss2), tuple(mm2)

        s8, m8 = lax.fori_loop(0, p1 - p0, piece_body, (s8, m8))
        fire(g + 2)
        return (s8, m8)

    lax.fori_loop(0, nch, chunk_body, ((zf,) * 8, (ninf,) * 8))

    wait_one()
    wait_one()

    pltpu.sync_copy(osum_v, out_s.at[pl.ds(n0, NPW)])
    pltpu.sync_copy(omax_v, out_m.at[pl.ds(n0, NPW)])

    cidx = idx_v.at[pl.ds(0, 64)]
    crow = rows_v.at[pl.ds(0, 64)]
    for t in range(NPW // 64):
        off = n0 + t * 64
        pltpu.sync_copy(c0i.at[pl.ds(off, 64)], cidx)
        pltpu.async_copy(h.at[cidx], crow, sem).wait()
        pltpu.sync_copy(crow, out_c0.at[pl.ds(off, 64)])
        pltpu.sync_copy(c1i.at[pl.ds(off, 64)], cidx)
        pltpu.async_copy(h.at[cidx], crow, sem).wait()
        pltpu.sync_copy(crow, out_c1.at[pl.ds(off, 64)])


_f32 = jnp.float32


@functools.cache
def _sc_agg_built():
    return functools.partial(
        pl.kernel,
        out_type=[jax.ShapeDtypeStruct((NPAD, H), _f32)] * 4,
        mesh=plsc.VectorSubcoreMesh(core_axis_name="c", subcore_axis_name="s"),
        scratch_types=[
            pltpu.VMEM((48,), jnp.int32),
            pltpu.VMEM((48,), jnp.int32),
            pltpu.VMEM((2 * CH,), jnp.int32),
            pltpu.VMEM((2 * CH, H), _f32),
            pltpu.VMEM((PF_W,), jnp.int32),
            pltpu.VMEM((PT_W,), jnp.int32),
            pltpu.VMEM((NPW, H), _f32),
            pltpu.VMEM((NPW, H), _f32),
            pltpu.SemaphoreType.DMA,
        ],
    )(_sc_agg_body)


def _sc_agg(*args):
    return _sc_agg_built()(*args)


BR = 1000


def _mm(a, b):
    return lax.dot_general(a, b, (((1,), (0,)), ((), ())),
                           precision=lax.Precision.DEFAULT,
                           preferred_element_type=_f32)


def _sigmoid(x):
    return 1.0 / (1.0 + jnp.exp(-x))


def _gru(x, h, wih, whh, bih, bhh):
    gi = _mm(x, wih) + bih
    gh = _mm(h, whh) + bhh
    r = _sigmoid(gi[:, 0:H] + gh[:, 0:H])
    z = _sigmoid(gi[:, H:2 * H] + gh[:, H:2 * H])
    nn_ = jnp.tanh(gi[:, 2 * H:3 * H] + r * gh[:, 2 * H:3 * H])
    return (1.0 - z) * nn_ + z * h


def _tc_round_body(h, sm, mx, c0, c1, meta, acw, acb, abw, abb,
                   w3i, w3h, bihc, bihb, bihl, bhhc, bhhb, bhhl,
                   lng, lnb, hout):
    cnt = meta[:, 0:1]
    invd = meta[:, 1:2]
    il = meta[:, 2:3]
    ic = meta[:, 3:4]
    ip = meta[:, 4:5]
    hv = h[...]
    mean = sm[...] * invd
    mxv = jnp.where(cnt > 0.0, mx[...], 0.0)
    aggc = _mm(jnp.concatenate([mean, mxv], axis=1), acw[...]) + acb[...]
    aggb = _mm(jnp.concatenate([c0[...], c1[...]], axis=1), abw[...]) + abb[...]
    agg = jnp.where(il > 0.0, 0.0,
                    jnp.where(ic > 0.0, aggc,
                              jnp.where(ip > 0.0, aggb, mean)))
    ml = il
    mb = ip
    mc = 1.0 - il - ip
    a3 = jnp.concatenate([agg * mc, agg * mb, agg * ml], axis=1)
    h3 = jnp.concatenate([hv * mc, hv * mb, hv * ml], axis=1)
    gi = _mm(a3, w3i[...]) + (mc * bihc[...] + mb * bihb[...]
                              + ml * bihl[...])
    gh = _mm(h3, w3h[...]) + (mc * bhhc[...] + mb * bhhb[...]
                              + ml * bhhl[...])
    r = _sigmoid(gi[:, 0:H] + gh[:, 0:H])
    z = _sigmoid(gi[:, H:2 * H] + gh[:, H:2 * H])
    nn_ = jnp.tanh(gi[:, 2 * H:3 * H] + r * gh[:, 2 * H:3 * H])
    upd = (1.0 - z) * nn_ + z * hv
    x = upd + hv
    mu = jnp.mean(x, axis=1, keepdims=True)
    var = jnp.mean((x - mu) * (x - mu), axis=1, keepdims=True)
    hout[...] = (x - mu) * lax.rsqrt(var + 1e-5) * lng[...] + lnb[...]


def _node_spec():
    return pl.BlockSpec((BR, H), lambda i: (i, 0))


def _full_spec(shape):
    return pl.BlockSpec(shape, lambda i: tuple(0 for _ in shape))


def _tc_round(h, sm, mx, c0, c1, meta, *weights):
    wspecs = [_full_spec(w.shape) for w in weights]
    return pl.pallas_call(
        _tc_round_body,
        grid=(N // BR,),
        in_specs=[_node_spec()] * 5 + [pl.BlockSpec((BR, 8), lambda i: (i, 0))]
                 + wspecs,
        out_specs=_node_spec(),
        out_shape=jax.ShapeDtypeStruct((N, H), _f32),
    )(h, sm, mx, c0, c1, meta, *weights)


def _tc_init_body(meta, ttab, ctab, vtab, iw, ib, hout):
    nt = meta[:, 0:1]
    ci = meta[:, 1:2]
    vi = meta[:, 2:3]
    oh_t = (nt == lax.broadcasted_iota(jnp.int32, (BR, 8), 1).astype(_f32)
            ).astype(_f32)
    te = _mm(oh_t, ttab[...])
    oh_c = (ci == lax.broadcasted_iota(jnp.int32, (BR, 24), 1).astype(_f32)
            ).astype(_f32)
    ce = _mm(oh_c, ctab[...])
    oh_v = (vi == lax.broadcasted_iota(jnp.int32, (BR, 8), 1).astype(_f32)
            ).astype(_f32)
    vee = _mm(oh_v, vtab[...])
    ve = jnp.where(nt == 3.0, ce, jnp.where(nt == 4.0, vee, 0.0))
    hout[...] = _mm(jnp.concatenate([te, ve], axis=1), iw[...]) + ib[...]


def _tc_init(meta, ttab, ctab, vtab, iw, ib):
    specs = [pl.BlockSpec((BR, 8), lambda i: (i, 0))]
    specs += [_full_spec(x.shape) for x in (ttab, ctab, vtab, iw, ib)]
    return pl.pallas_call(
        _tc_init_body,
        grid=(N // BR,),
        in_specs=specs,
        out_specs=_node_spec(),
        out_shape=jax.ShapeDtypeStruct((N, H), _f32),
    )(meta, ttab, ctab, vtab, iw, ib)


def _tc_final_body(h, gw, gb, emb, gco, gc_v):
    i = pl.program_id(0)

    @pl.when(i == 0)
    def _():
        gc_v[...] = _mm(h[0:8, :], gw[...]) + gb[...]

    hv = h[...]
    emb[:, 0:H] = hv
    emb[:, H:2 * H] = jnp.broadcast_to(gc_v[0:1, :], (BR, H))
    gco[...] = gc_v[...]


def _tc_final(h, gw, gb):
    return pl.pallas_call(
        _tc_final_body,
        grid=(N // BR,),
        in_specs=[_node_spec(), _full_spec((H, H)), _full_spec((1, H))],
        out_specs=[pl.BlockSpec((BR, 2 * H), lambda i: (i, 0)),
                   _full_spec((8, H))],
        out_shape=[jax.ShapeDtypeStruct((N, 2 * H), _f32),
                   jax.ShapeDtypeStruct((8, H), _f32)],
        scratch_shapes=[pltpu.VMEM((8, H), _f32)],
    )(h, gw, gb)


def _edge_tables(src, dst):
    i32 = jnp.int32
    start = jnp.searchsorted(dst, jnp.arange(N + 1, dtype=i32), side='left')
    start = start.astype(i32)
    bounds = jnp.minimum(jnp.arange(33, dtype=i32) * NPW, N)
    eoff = start[bounds]
    eoffp = jnp.concatenate([eoff, jnp.full((15,), E, i32)])
    srcp = jnp.concatenate([src, jnp.zeros((3 * CH,), i32)])

    eidx = jnp.arange(E, dtype=i32)
    w_of_e = dst // NPW
    wch = jnp.concatenate([jnp.ones((1,), jnp.bool_),
                           w_of_e[1:] != w_of_e[:E - 1]])
    e0_e = lax.cummax(jnp.where(wch, eidx, 0))
    a_e = e0_e % 8
    e0a_e = e0_e - a_e
    prev = jnp.concatenate([jnp.full((1,), -1, i32), dst[:E - 1]])
    newp = (dst != prev) | wch | (((eidx - e0_e + a_e) % CH) == 0)
    pid = jnp.cumsum(newp.astype(i32)).astype(i32) - 1
    ptot = pid[E - 1] + 1

    ps_next = jnp.concatenate([jnp.where(newp, eidx, E)[1:],
                               jnp.full((1,), E, i32)])
    nps = lax.cummin(ps_next, reverse=True)
    lastn = jnp.concatenate([dst[1:] != dst[:E - 1],
                             jnp.ones((1,), jnp.bool_)])
    nlast = lax.cummin(jnp.where(lastn, eidx, E), reverse=True)
    take = nps - eidx
    flush = (nlast < nps).astype(i32)
    lnode = dst % NPW
    row = (((eidx - e0a_e) // CH) % 2) * CH + (eidx - e0a_e) % CH
    packed = ((take & 0xFF) | ((row & 0xFF) << 8) | ((lnode & 0x1FF) << 16)
              | (flush << 26))
    key = jnp.where(newp, pid, jnp.int32(2 ** 30))
    _, sv, sef = lax.sort((key, packed, eidx), num_keys=1)
    ptabg = sv[:PT_G]
    ef = jnp.where(jnp.arange(PT_G, dtype=i32) < ptot, sef[:PT_G], E)

    e0w = eoff[:32]
    e0aw = e0w - e0w % 8
    nchw = (eoff[1:] - e0aw + CH - 1) // CH
    cbase = jnp.concatenate([jnp.zeros((1,), i32),
                             jnp.cumsum(nchw).astype(i32)])
    cbp = jnp.concatenate([cbase, jnp.full((15,), cbase[32], i32)])
    marks = jnp.clip(cbase[:32], 0, CT_G - 1)
    e0q = lax.cummax(jnp.zeros((CT_G,), i32).at[marks].max(e0w))
    cbw = lax.cummax(jnp.zeros((CT_G,), i32).at[marks].max(cbase[:32]))
    cq = jnp.arange(CT_G, dtype=i32)
    sposq = jnp.maximum(e0q, (e0q - e0q % 8) + (cq - cbw) * CH)
    poffg = jnp.where(cq >= cbase[32], ptot,
                      jnp.searchsorted(ef, jnp.clip(sposq, 0, E - 1),
                                       side='left').astype(i32))
    return start, eoffp, srcp, cbp, poffg, ptabg


def kernel(node_type, coeff_idx, var_idx, src, dst, type_table, coeff_table,
           var_table, init_w, init_b, agg_comm_w, agg_comm_b, agg_bin_w,
           agg_bin_b, wih_comm, whh_comm, bih_comm, bhh_comm, wih_bin,
           whh_bin, bih_bin, bhh_bin, wih_leaf, whh_leaf, bih_leaf, bhh_leaf,
           ln_g, ln_b, glob_w, glob_b):
    i32 = jnp.int32
    src = src.astype(i32)
    dst = dst.astype(i32)

    start, eoffp, srcp, cbp, poffg, ptabg = _edge_tables(src, dst)
    counts = (start[1:] - start[:N]).astype(_f32)
    c0 = src[jnp.clip(start[:N], 0, E - 1)]
    c1 = src[jnp.clip(start[:N] + 1, 0, E - 1)]
    c0p = jnp.concatenate([c0, jnp.zeros((NPAD - N,), i32)])
    c1p = jnp.concatenate([c1, jnp.zeros((NPAD - N,), i32)])

    is_leaf = (counts == 0.0)
    is_comm = (node_type <= 1) & (~is_leaf)
    is_pow2 = (node_type == 2) & (counts == 2.0)
    invd = 1.0 / jnp.maximum(counts, 1.0)
    meta = jnp.stack([counts, invd,
                      is_leaf.astype(_f32), is_comm.astype(_f32),
                      is_pow2.astype(_f32),
                      jnp.zeros((N,), _f32), jnp.zeros((N,), _f32),
                      jnp.zeros((N,), _f32)], axis=1)

    meta_init = jnp.stack([node_type.astype(_f32), coeff_idx.astype(_f32),
                           var_idx.astype(_f32)] + [jnp.zeros((N,), _f32)] * 5,
                          axis=1)
    ttab = jnp.zeros((8, H), _f32).at[:6].set(type_table)
    ctab = jnp.zeros((24, H), _f32).at[:19].set(coeff_table)
    vtab = jnp.zeros((8, H), _f32).at[:4].set(var_table)

    w3i = jnp.concatenate([wih_comm, wih_bin, wih_leaf], axis=0)
    w3h = jnp.concatenate([whh_comm, whh_bin, whh_leaf], axis=0)
    rw = [agg_comm_w, agg_comm_b.reshape(1, H), agg_bin_w,
          agg_bin_b.reshape(1, H),
          w3i, w3h,
          bih_comm.reshape(1, 3 * H), bih_bin.reshape(1, 3 * H),
          bih_leaf.reshape(1, 3 * H),
          bhh_comm.reshape(1, 3 * H), bhh_bin.reshape(1, 3 * H),
          bhh_leaf.reshape(1, 3 * H),
          ln_g.reshape(1, H), ln_b.reshape(1, H)]

    h = _tc_init(meta_init, ttab, ctab, vtab, init_w, init_b.reshape(1, H))
    for _r in range(NUM_ROUNDS):
        sm, mx, hc0, hc1 = _sc_agg(h, srcp, eoffp, cbp, poffg, ptabg,
                                   c0p, c1p)
        h = _tc_round(h, sm, mx, hc0, hc1, meta, *rw)

    emb, gc8 = _tc_final(h, glob_w, glob_b.reshape(1, H))
    return emb, gc8[0]


# speedup vs baseline: 2.7878x; 1.0064x over previous
import functools

import jax
import jax.numpy as jnp
from jax import lax
from jax.experimental import pallas as pl
from jax.experimental.pallas import tpu as pltpu
from jax.experimental.pallas import tpu_sc as plsc

N = 10000
E = 320000
H = 128
NUM_ROUNDS = 6

NW = 32
NPW = 320
NPAD = NW * NPW
CH = 128
EPAD = E + 3 * CH
PF_W = 2576
PT_W = 2896
CT_G = 5248
PT_G = 16384


def _sc_agg_body(h, srcp, eoffp, cbp, poffg, ptabg, c0i, c1i,
                 out_s, out_m, out_c0, out_c1,
                 eoff_v, cb_v, idx_v, rows_v, poff_v, ptab_v, osum_v, omax_v,
                 sem):
    c = lax.axis_index("c")
    s = lax.axis_index("s")
    w = c * 16 + s
    n0 = w * NPW

    pltpu.sync_copy(eoffp, eoff_v)
    pltpu.sync_copy(cbp, cb_v)
    ev = eoff_v[pl.ds(w, 16)]
    e0 = ev[0]
    e0a = (e0 // 8) * 8
    cv = cb_v[pl.ds(w, 16)]
    cb = cv[0]
    nch = cv[1] - cb

    cba = (cb // 8) * 8
    dcb = cb - cba
    pltpu.sync_copy(poffg.at[pl.ds(cba, PF_W)], poff_v)
    pv = poff_v[pl.ds(dcb, 16)]
    pb = pv[0]
    pba = (pb // 8) * 8
    dpb = pb - pba
    pltpu.sync_copy(ptabg.at[pl.ds(pba, PT_W)], ptab_v)

    def fire(g):
        slot = (g % 2) * CH
        pltpu.sync_copy(srcp.at[pl.ds(e0a + g * CH, CH)],
                        idx_v.at[pl.ds(slot, CH)])
        pltpu.async_copy(h.at[idx_v.at[pl.ds(slot, CH)]],
                         rows_v.at[pl.ds(slot, CH)], sem)

    def wait_one():
        pltpu.make_async_copy(h.at[pl.ds(0, CH)],
                              rows_v.at[pl.ds(0, CH)], sem).wait()

    fire(0)
    fire(1)

    zf = jnp.zeros((16,), jnp.float32)
    ninf = jnp.full((16,), -jnp.inf, jnp.float32)

    def init_body(i, carry):
        for j in range(8):
            osum_v[i, pl.ds(j * 16, 16)] = zf
            omax_v[i, pl.ds(j * 16, 16)] = ninf
        return carry

    lax.fori_loop(0, NPW, init_body, 0)

    def chunk_body(g, carry):
        s8, m8 = carry
        wait_one()
        pvec = poff_v[pl.ds(dcb + g, 16)]
        p0 = pvec[0]
        p1 = pvec[1]

        def piece_body(k, carry2):
            ss, mm = carry2
            fvec = ptab_v[pl.ds(dpb + (p0 - pb) + k, 16)]
            f = fvec[0]
            take = f & 0xFF
            row0 = (f >> 8) & 0xFF
            ln = (f >> 16) & 0x1FF
            flush = (f >> 26) & 1

            def inner(e, sm):
                ss3, mm3 = sm
                r = row0 + e
                ss4 = []
                mm4 = []
                for j in range(8):
                    row = rows_v[r, pl.ds(j * 16, 16)]
                    ss4.append(ss3[j] + row)
                    mm4.append(jnp.maximum(mm3[j], row))
                return tuple(ss4), tuple(mm4)

            ss, mm = lax.fori_loop(0, take, inner, (ss, mm))
            keepf = jnp.full((16,), 1.0, jnp.float32) * (1 - flush).astype(
                jnp.float32)
            clamp = (keepf * 2.0 - 1.0) * jnp.full((16,), jnp.inf, jnp.float32)
            ss2 = []
            mm2 = []
            for j in range(8):
                osum_v[ln, pl.ds(j * 16, 16)] = ss[j]
                omax_v[ln, pl.ds(j * 16, 16)] = mm[j]
                ss2.append(ss[j] * keepf)
                mm2.append(jnp.minimum(mm[j], clamp))
            return tuple(ss2), tuple(mm2)

        s8, m8 = lax.fori_loop(0, p1 - p0, piece_body, (s8, m8))
        fire(g + 2)
        return (s8, m8)

    lax.fori_loop(0, nch, chunk_body, ((zf,) * 8, (ninf,) * 8))

    wait_one()
    wait_one()

    pltpu.sync_copy(osum_v, out_s.at[pl.ds(n0, NPW)])
    pltpu.sync_copy(omax_v, out_m.at[pl.ds(n0, NPW)])

    cidx = idx_v.at[pl.ds(0, 64)]
    crow = rows_v.at[pl.ds(0, 64)]
    for t in range(NPW // 64):
        off = n0 + t * 64
        pltpu.sync_copy(c0i.at[pl.ds(off, 64)], cidx)
        pltpu.async_copy(h.at[cidx], crow, sem).wait()
        pltpu.sync_copy(crow, out_c0.at[pl.ds(off, 64)])
        pltpu.sync_copy(c1i.at[pl.ds(off, 64)], cidx)
        pltpu.async_copy(h.at[cidx], crow, sem).wait()
        pltpu.sync_copy(crow, out_c1.at[pl.ds(off, 64)])


_f32 = jnp.float32


@functools.cache
def _sc_agg_built():
    return functools.partial(
        pl.kernel,
        out_type=[jax.ShapeDtypeStruct((NPAD, H), _f32)] * 4,
        mesh=plsc.VectorSubcoreMesh(core_axis_name="c", subcore_axis_name="s"),
        scratch_types=[
            pltpu.VMEM((48,), jnp.int32),
            pltpu.VMEM((48,), jnp.int32),
            pltpu.VMEM((2 * CH,), jnp.int32),
            pltpu.VMEM((2 * CH, H), _f32),
            pltpu.VMEM((PF_W,), jnp.int32),
            pltpu.VMEM((PT_W,), jnp.int32),
            pltpu.VMEM((NPW, H), _f32),
            pltpu.VMEM((NPW, H), _f32),
            pltpu.SemaphoreType.DMA,
        ],
    )(_sc_agg_body)


def _sc_agg(*args):
    return _sc_agg_built()(*args)


BR = 1000


def _mm(a, b):
    return lax.dot_general(a, b, (((1,), (0,)), ((), ())),
                           precision=lax.Precision.DEFAULT,
                           preferred_element_type=_f32)


def _sigmoid(x):
    return 1.0 / (1.0 + jnp.exp(-x))


def _gru(x, h, wih, whh, bih, bhh):
    gi = _mm(x, wih) + bih
    gh = _mm(h, whh) + bhh
    r = _sigmoid(gi[:, 0:H] + gh[:, 0:H])
    z = _sigmoid(gi[:, H:2 * H] + gh[:, H:2 * H])
    nn_ = jnp.tanh(gi[:, 2 * H:3 * H] + r * gh[:, 2 * H:3 * H])
    return (1.0 - z) * nn_ + z * h


def _tc_round_body(h, sm, mx, c0, c1, meta, acw, acb, abw, abb,
                   w3i, w3h, bihc, bihb, bihl, bhhc, bhhb, bhhl,
                   lng, lnb, hout):
    cnt = meta[:, 0:1]
    invd = meta[:, 1:2]
    il = meta[:, 2:3]
    ic = meta[:, 3:4]
    ip = meta[:, 4:5]
    hv = h[...]
    mean = sm[...] * invd
    mxv = jnp.where(cnt > 0.0, mx[...], 0.0)
    aggc = _mm(jnp.concatenate([mean, mxv], axis=1), acw[...]) + acb[...]
    aggb = _mm(jnp.concatenate([c0[...], c1[...]], axis=1), abw[...]) + abb[...]
    agg = jnp.where(il > 0.0, 0.0,
                    jnp.where(ic > 0.0, aggc,
                              jnp.where(ip > 0.0, aggb, mean)))
    ml = il
    mb = ip
    mc = 1.0 - il - ip
    a3 = jnp.concatenate([agg * mc, agg * mb, agg * ml], axis=1)
    h3 = jnp.concatenate([hv * mc, hv * mb, hv * ml], axis=1)
    gi = _mm(a3, w3i[...]) + (mc * bihc[...] + mb * bihb[...]
                              + ml * bihl[...])
    gh = _mm(h3, w3h[...]) + (mc * bhhc[...] + mb * bhhb[...]
                              + ml * bhhl[...])
    r = _sigmoid(gi[:, 0:H] + gh[:, 0:H])
    z = _sigmoid(gi[:, H:2 * H] + gh[:, H:2 * H])
    nn_ = jnp.tanh(gi[:, 2 * H:3 * H] + r * gh[:, 2 * H:3 * H])
    upd = (1.0 - z) * nn_ + z * hv
    x = upd + hv
    mu = jnp.mean(x, axis=1, keepdims=True)
    var = jnp.mean((x - mu) * (x - mu), axis=1, keepdims=True)
    hout[...] = (x - mu) * lax.rsqrt(var + 1e-5) * lng[...] + lnb[...]


def _node_spec():
    return pl.BlockSpec((BR, H), lambda i: (i, 0))


def _full_spec(shape):
    return pl.BlockSpec(shape, lambda i: tuple(0 for _ in shape))


def _tc_round(h, sm, mx, c0, c1, meta, *weights):
    wspecs = [_full_spec(w.shape) for w in weights]
    return pl.pallas_call(
        _tc_round_body,
        grid=(N // BR,),
        in_specs=[_node_spec()] * 5 + [pl.BlockSpec((BR, 8), lambda i: (i, 0))]
                 + wspecs,
        out_specs=_node_spec(),
        out_shape=jax.ShapeDtypeStruct((N, H), _f32),
    )(h, sm, mx, c0, c1, meta, *weights)


def _tc_init_body(meta, ttab, ctab, vtab, iw, ib, hout):
    nt = meta[:, 0:1]
    ci = meta[:, 1:2]
    vi = meta[:, 2:3]
    oh_t = (nt == lax.broadcasted_iota(jnp.int32, (BR, 8), 1).astype(_f32)
            ).astype(_f32)
    te = _mm(oh_t, ttab[...])
    oh_c = (ci == lax.broadcasted_iota(jnp.int32, (BR, 24), 1).astype(_f32)
            ).astype(_f32)
    ce = _mm(oh_c, ctab[...])
    oh_v = (vi == lax.broadcasted_iota(jnp.int32, (BR, 8), 1).astype(_f32)
            ).astype(_f32)
    vee = _mm(oh_v, vtab[...])
    ve = jnp.where(nt == 3.0, ce, jnp.where(nt == 4.0, vee, 0.0))
    hout[...] = _mm(jnp.concatenate([te, ve], axis=1), iw[...]) + ib[...]


def _tc_init(meta, ttab, ctab, vtab, iw, ib):
    specs = [pl.BlockSpec((BR, 8), lambda i: (i, 0))]
    specs += [_full_spec(x.shape) for x in (ttab, ctab, vtab, iw, ib)]
    return pl.pallas_call(
        _tc_init_body,
        grid=(N // BR,),
        in_specs=specs,
        out_specs=_node_spec(),
        out_shape=jax.ShapeDtypeStruct((N, H), _f32),
    )(meta, ttab, ctab, vtab, iw, ib)


def _tc_final_body(h, gw, gb, emb, gco, gc_v):
    i = pl.program_id(0)

    @pl.when(i == 0)
    def _():
        gc_v[...] = _mm(h[0:8, :], gw[...]) + gb[...]

    hv = h[...]
    emb[:, 0:H] = hv
    emb[:, H:2 * H] = jnp.broadcast_to(gc_v[0:1, :], (BR, H))
    gco[...] = gc_v[...]


def _tc_final(h, gw, gb):
    return pl.pallas_call(
        _tc_final_body,
        grid=(N // BR,),
        in_specs=[_node_spec(), _full_spec((H, H)), _full_spec((1, H))],
        out_specs=[pl.BlockSpec((BR, 2 * H), lambda i: (i, 0)),
                   _full_spec((8, H))],
        out_shape=[jax.ShapeDtypeStruct((N, 2 * H), _f32),
                   jax.ShapeDtypeStruct((8, H), _f32)],
        scratch_shapes=[pltpu.VMEM((8, H), _f32)],
    )(h, gw, gb)


def _edge_tables(src, dst):
    i32 = jnp.int32
    start = jnp.searchsorted(dst, jnp.arange(N + 1, dtype=i32), side='left')
    start = start.astype(i32)
    bounds = jnp.minimum(jnp.arange(33, dtype=i32) * NPW, N)
    eoff = start[bounds]
    eoffp = jnp.concatenate([eoff, jnp.full((15,), E, i32)])
    srcp = jnp.concatenate([src, jnp.zeros((3 * CH,), i32)])

    eidx = jnp.arange(E, dtype=i32)
    w_of_e = dst // NPW
    wch = jnp.concatenate([jnp.ones((1,), jnp.bool_),
                           w_of_e[1:] != w_of_e[:E - 1]])
    e0_e = lax.cummax(jnp.where(wch, eidx, 0))
    a_e = e0_e % 8
    e0a_e = e0_e - a_e
    prev = jnp.concatenate([jnp.full((1,), -1, i32), dst[:E - 1]])
    newp = (dst != prev) | wch | (((eidx - e0_e + a_e) % CH) == 0)
    pid = jnp.cumsum(newp.astype(i32)).astype(i32) - 1
    ptot = pid[E - 1] + 1

    ps_next = jnp.concatenate([jnp.where(newp, eidx, E)[1:],
                               jnp.full((1,), E, i32)])
    nps = lax.cummin(ps_next, reverse=True)
    lastn = jnp.concatenate([dst[1:] != dst[:E - 1],
                             jnp.ones((1,), jnp.bool_)])
    nlast = lax.cummin(jnp.where(lastn, eidx, E), reverse=True)
    take = nps - eidx
    flush = (nlast < nps).astype(i32)
    lnode = dst % NPW
    row = (((eidx - e0a_e) // CH) % 2) * CH + (eidx - e0a_e) % CH
    packed = ((take & 0xFF) | ((row & 0xFF) << 8) | ((lnode & 0x1FF) << 16)
              | (flush << 26))
    key = jnp.where(newp, pid, jnp.int32(2 ** 30))
    _, sv, sef = lax.sort((key, packed, eidx), num_keys=1)
    ptabg = sv[:PT_G]
    ef = jnp.where(jnp.arange(PT_G, dtype=i32) < ptot, sef[:PT_G], E)

    e0w = eoff[:32]
    e0aw = e0w - e0w % 8
    nchw = (eoff[1:] - e0aw + CH - 1) // CH
    cbase = jnp.concatenate([jnp.zeros((1,), i32),
                             jnp.cumsum(nchw).astype(i32)])
    cbp = jnp.concatenate([cbase, jnp.full((15,), cbase[32], i32)])
    marks = jnp.clip(cbase[:32], 0, CT_G - 1)
    e0q = lax.cummax(jnp.zeros((CT_G,), i32).at[marks].max(e0w))
    cbw = lax.cummax(jnp.zeros((CT_G,), i32).at[marks].max(cbase[:32]))
    cq = jnp.arange(CT_G, dtype=i32)
    sposq = jnp.maximum(e0q, (e0q - e0q % 8) + (cq - cbw) * CH)
    poffg = jnp.where(cq >= cbase[32], ptot,
                      jnp.searchsorted(ef, jnp.clip(sposq, 0, E - 1),
                                       side='left').astype(i32))
    return start, eoffp, srcp, cbp, poffg, ptabg


def kernel(node_type, coeff_idx, var_idx, src, dst, type_table, coeff_table,
           var_table, init_w, init_b, agg_comm_w, agg_comm_b, agg_bin_w,
           agg_bin_b, wih_comm, whh_comm, bih_comm, bhh_comm, wih_bin,
           whh_bin, bih_bin, bhh_bin, wih_leaf, whh_leaf, bih_leaf, bhh_leaf,
           ln_g, ln_b, glob_w, glob_b):
    i32 = jnp.int32
    src = src.astype(i32)
    dst = dst.astype(i32)

    start, eoffp, srcp, cbp, poffg, ptabg = _edge_tables(src, dst)
    counts = (start[1:] - start[:N]).astype(_f32)
    srcnext = jnp.concatenate([src[1:], jnp.zeros((1,), i32)])
    spk = src | (srcnext << 16)
    c01 = spk[jnp.clip(start[:N], 0, E - 1)]
    c0 = c01 & 0xFFFF
    c1 = (c01 >> 16) & 0xFFFF
    c0p = jnp.concatenate([c0, jnp.zeros((NPAD - N,), i32)])
    c1p = jnp.concatenate([c1, jnp.zeros((NPAD - N,), i32)])

    is_leaf = (counts == 0.0)
    is_comm = (node_type <= 1) & (~is_leaf)
    is_pow2 = (node_type == 2) & (counts == 2.0)
    invd = 1.0 / jnp.maximum(counts, 1.0)
    meta = jnp.stack([counts, invd,
                      is_leaf.astype(_f32), is_comm.astype(_f32),
                      is_pow2.astype(_f32),
                      jnp.zeros((N,), _f32), jnp.zeros((N,), _f32),
                      jnp.zeros((N,), _f32)], axis=1)

    meta_init = jnp.stack([node_type.astype(_f32), coeff_idx.astype(_f32),
                           var_idx.astype(_f32)] + [jnp.zeros((N,), _f32)] * 5,
                          axis=1)
    ttab = jnp.zeros((8, H), _f32).at[:6].set(type_table)
    ctab = jnp.zeros((24, H), _f32).at[:19].set(coeff_table)
    vtab = jnp.zeros((8, H), _f32).at[:4].set(var_table)

    w3i = jnp.concatenate([wih_comm, wih_bin, wih_leaf], axis=0)
    w3h = jnp.concatenate([whh_comm, whh_bin, whh_leaf], axis=0)
    rw = [agg_comm_w, agg_comm_b.reshape(1, H), agg_bin_w,
          agg_bin_b.reshape(1, H),
          w3i, w3h,
          bih_comm.reshape(1, 3 * H), bih_bin.reshape(1, 3 * H),
          bih_leaf.reshape(1, 3 * H),
          bhh_comm.reshape(1, 3 * H), bhh_bin.reshape(1, 3 * H),
          bhh_leaf.reshape(1, 3 * H),
          ln_g.reshape(1, H), ln_b.reshape(1, H)]

    h = _tc_init(meta_init, ttab, ctab, vtab, init_w, init_b.reshape(1, H))
    for _r in range(NUM_ROUNDS):
        sm, mx, hc0, hc1 = _sc_agg(h, srcp, eoffp, cbp, poffg, ptabg,
                                   c0p, c1p)
        h = _tc_round(h, sm, mx, hc0, hc1, meta, *rw)

    emb, gc8 = _tc_final(h, glob_w, glob_b.reshape(1, H))
    return emb, gc8[0]
